# baseline (speedup 1.0000x reference)
import jax
import jax.numpy as jnp
from jax import lax
from jax.experimental import pallas as pl
from jax.experimental.pallas import tpu as pltpu
from jax.experimental.pallas import tpu_sc as plsc

B = 16384
NC, NS, L = 2, 16, 16
NW = NC * NS
BPW = B // NW
CH = 128
RPW = BPW // CH
NU = 6
NUSER = 1000000
NMOVIE = 100000
TCH = 25000
NUC = NUSER // TCH
NMC = NMOVIE // TCH

_mesh = plsc.VectorSubcoreMesh(core_axis_name="c", subcore_axis_name="s",
                               num_cores=NC, num_subcores=NS)


@pl.kernel(
    out_type=jax.ShapeDtypeStruct((B,), jnp.float32),
    mesh=_mesh,
    scratch_types=[
        pltpu.VMEM((NU * RPW, CH), jnp.int32),
        pltpu.VMEM((RPW, CH), jnp.int32),
        pltpu.VMEM((NU * RPW, CH), jnp.float32),
        pltpu.VMEM((RPW, CH), jnp.float32),
        pltpu.VMEM((BPW,), jnp.float32),
        pltpu.SemaphoreType.DMA,
    ],
    compiler_params=pltpu.CompilerParams(use_tc_tiling_on_sc=False),
)
def _first_order_sc(uidx_hbm, midx_hbm, wu_hbm, wm_hbm, out_hbm,
                    uidx_v, midx_v, urows, mrows, out_v, sem):
    wid = lax.axis_index("s") * NC + lax.axis_index("c")
    base_r = wid * RPW

    for j in range(NU):
        pltpu.sync_copy(uidx_hbm.at[pl.ds(j * (B // CH) + base_r, RPW)],
                        uidx_v.at[pl.ds(j * RPW, RPW)])
    pltpu.sync_copy(midx_hbm.at[pl.ds(base_r, RPW)], midx_v)

    copies = []
    for t in range(NU * RPW):
        copies.append(pltpu.async_copy(wu_hbm.at[uidx_v.at[pl.ds(t, 1)]],
                                       urows.at[pl.ds(t, 1)], sem))
    for t in range(RPW):
        copies.append(pltpu.async_copy(wm_hbm.at[midx_v.at[pl.ds(t, 1)]],
                                       mrows.at[pl.ds(t, 1)], sem))
    for cp in copies:
        cp.wait()

    for l in range(BPW // L):
        row, col = l // (CH // L), (l % (CH // L)) * L
        acc = mrows[row, pl.ds(col, L)]
        for j in range(NU):
            acc = acc + urows[j * RPW + row, pl.ds(col, L)]
        out_v[pl.ds(l * L, L)] = acc

    pltpu.sync_copy(out_v, out_hbm.at[pl.ds(wid * BPW, BPW)])


def kernel(users, movies, gens, W_user, W_movie, W_genere):
    del W_genere
    uidx = jnp.concatenate(
        [users.astype(jnp.int32)[None, :], gens.astype(jnp.int32).T], axis=0)
    uidx = uidx.reshape(NU * B // CH, CH)
    midx = movies.astype(jnp.int32).reshape(B // CH, CH)
    return _first_order_sc(uidx, midx, W_user.reshape(1, -1),
                           W_movie.reshape(1, -1))


# speedup vs baseline: 1.0474x; 1.0474x over previous
import jax
import jax.numpy as jnp
from jax import lax
from jax.experimental import pallas as pl
from jax.experimental.pallas import tpu as pltpu
from jax.experimental.pallas import tpu_sc as plsc

B = 16384
NC, NS, L = 2, 16, 16
NW = NC * NS
BPW = B // NW
CH = 512
RPW = BPW // CH
NU = 6
NUSER = 1000000
NMOVIE = 100000
TCH = 25000
NUC = NUSER // TCH
NMC = NMOVIE // TCH

_mesh = plsc.VectorSubcoreMesh(core_axis_name="c", subcore_axis_name="s",
                               num_cores=NC, num_subcores=NS)


@pl.kernel(
    out_type=jax.ShapeDtypeStruct((B,), jnp.float32),
    mesh=_mesh,
    scratch_types=[
        pltpu.VMEM((NU * RPW, CH), jnp.int32),
        pltpu.VMEM((RPW, CH), jnp.int32),
        pltpu.VMEM((NU * RPW, CH), jnp.float32),
        pltpu.VMEM((RPW, CH), jnp.float32),
        pltpu.SemaphoreType.DMA,
        pltpu.VMEM((BPW,), jnp.float32),
        pltpu.SemaphoreType.DMA,
    ],
    compiler_params=pltpu.CompilerParams(use_tc_tiling_on_sc=False),
)
def _first_order_sc(uidx_hbm, midx_hbm, wu_hbm, wm_hbm, out_hbm,
                    uidx_v, midx_v, urows, mrows, idx_sem, out_v, sem):
    wid = lax.axis_index("s") * NC + lax.axis_index("c")
    base_r = wid * RPW

    stages = []
    for j in range(NU):
        stages.append(pltpu.async_copy(
            uidx_hbm.at[pl.ds(j * (B // CH) + base_r, RPW)],
            uidx_v.at[pl.ds(j * RPW, RPW)], idx_sem))
    stages.append(pltpu.async_copy(midx_hbm.at[pl.ds(base_r, RPW)], midx_v,
                                   idx_sem))
    for st in stages:
        st.wait()

    copies = []
    for t in range(NU * RPW):
        copies.append(pltpu.async_copy(wu_hbm.at[uidx_v.at[pl.ds(t, 1)]],
                                       urows.at[pl.ds(t, 1)], sem))
    for t in range(RPW):
        copies.append(pltpu.async_copy(wm_hbm.at[midx_v.at[pl.ds(t, 1)]],
                                       mrows.at[pl.ds(t, 1)], sem))
    for cp in copies:
        cp.wait()

    for l in range(BPW // L):
        row, col = l // (CH // L), (l % (CH // L)) * L
        acc = mrows[row, pl.ds(col, L)]
        for j in range(NU):
            acc = acc + urows[j * RPW + row, pl.ds(col, L)]
        out_v[pl.ds(l * L, L)] = acc

    pltpu.sync_copy(out_v, out_hbm.at[pl.ds(wid * BPW, BPW)])


def kernel(users, movies, gens, W_user, W_movie, W_genere):
    del W_genere
    uidx = jnp.concatenate(
        [users.astype(jnp.int32)[None, :], gens.astype(jnp.int32).T], axis=0)
    uidx = uidx.reshape(NU * B // CH, CH)
    midx = movies.astype(jnp.int32).reshape(B // CH, CH)
    return _first_order_sc(uidx, midx, W_user.reshape(1, -1),
                           W_movie.reshape(1, -1))


# speedup vs baseline: 1.0475x; 1.0001x over previous
import jax
import jax.numpy as jnp
from jax import lax
from jax.experimental import pallas as pl
from jax.experimental.pallas import tpu as pltpu
from jax.experimental.pallas import tpu_sc as plsc

B = 16384
NC, NS, L = 2, 16, 16
NW = NC * NS
BPW = B // NW
CH = 512
RPW = BPW // CH
NU = 6

_mesh = plsc.VectorSubcoreMesh(core_axis_name="c", subcore_axis_name="s",
                               num_cores=NC, num_subcores=NS)


@pl.kernel(
    out_type=jax.ShapeDtypeStruct((B,), jnp.float32),
    mesh=_mesh,
    scratch_types=[
        pltpu.VMEM((NU * RPW, CH), jnp.int32),
        pltpu.VMEM((RPW, CH), jnp.int32),
        pltpu.VMEM((NU * RPW, CH), jnp.float32),
        pltpu.VMEM((RPW, CH), jnp.float32),
        pltpu.SemaphoreType.DMA,
        pltpu.VMEM((BPW,), jnp.float32),
        pltpu.SemaphoreType.DMA,
    ],
    compiler_params=pltpu.CompilerParams(use_tc_tiling_on_sc=False),
)
def _first_order_sc(uidx_hbm, midx_hbm, wu_hbm, wm_hbm, out_hbm,
                    uidx_v, midx_v, urows, mrows, idx_sem, out_v, sem):
    wid = lax.axis_index("s") * NC + lax.axis_index("c")
    base_r = wid * RPW

    stages = []
    for j in range(NU):
        stages.append(pltpu.async_copy(
            uidx_hbm.at[pl.ds(j * (B // CH) + base_r, RPW)],
            uidx_v.at[pl.ds(j * RPW, RPW)], idx_sem))
    stages.append(pltpu.async_copy(midx_hbm.at[pl.ds(base_r, RPW)], midx_v,
                                   idx_sem))
    for st in stages:
        st.wait()

    copies = []
    for t in range(NU * RPW):
        copies.append(pltpu.async_copy(wu_hbm.at[uidx_v.at[pl.ds(t, 1)]],
                                       urows.at[pl.ds(t, 1)], sem))
    for t in range(RPW):
        copies.append(pltpu.async_copy(wm_hbm.at[midx_v.at[pl.ds(t, 1)]],
                                       mrows.at[pl.ds(t, 1)], sem))
    for cp in copies:
        cp.wait()

    for l in range(BPW // L):
        row, col = l // (CH // L), (l % (CH // L)) * L
        acc = mrows[row, pl.ds(col, L)]
        for j in range(NU):
            acc = acc + urows[j * RPW + row, pl.ds(col, L)]
        out_v[pl.ds(l * L, L)] = acc

    pltpu.sync_copy(out_v, out_hbm.at[pl.ds(wid * BPW, BPW)])


def kernel(users, movies, gens, W_user, W_movie, W_genere):
    del W_genere
    uidx = jnp.concatenate(
        [users.astype(jnp.int32)[None, :], gens.astype(jnp.int32).T], axis=0)
    uidx = uidx.reshape(NU * B // CH, CH)
    midx = movies.astype(jnp.int32).reshape(B // CH, CH)
    return _first_order_sc(uidx, midx, W_user.reshape(1, -1),
                           W_movie.reshape(1, -1))
